# baseline (device time: 34959 ns/iter reference)
import jax
import jax.numpy as jnp
from jax import lax
from jax.experimental import pallas as pl
from jax.experimental.pallas import tpu as pltpu

N_DEV = 16
PLANE = 4
N_Z = 4
N_TOK = 1024
D_IN = 512
D_OUT = 1024
E_LOCAL = 4
ROWS = N_TOK // N_DEV


def kernel(x, router_W, route_idx, expert_W, shared_W):
    def body(x_ref, rw_ref, ri_ref, ew_ref, sw_ref, out_ref,
             partial_ref, partial_bf_ref, rp1_ref, rp2_ref, scale_ref,
             ew_bf_ref, p1_send_sems, p1_recv_sems, p2_send_sems, p2_recv_sems):
        d = lax.axis_index("i")
        my_z = lax.div(d, PLANE)
        my_w = lax.rem(d, PLANE)

        scores = x_ref[:, :] @ rw_ref[:, :]
        m = jnp.max(scores, axis=-1, keepdims=True)
        p = jnp.exp(scores - m)
        probs = p / jnp.sum(p, axis=-1, keepdims=True)
        route = ri_ref[:, :]
        eids = lax.broadcasted_iota(route.dtype, scores.shape, 1)
        coef = jnp.sum(jnp.where(eids == route, probs, 0.0),
                       axis=-1, keepdims=True)
        scale_ref[:, :] = jnp.concatenate(
            [jnp.where(route == d * E_LOCAL + e, coef, 0.0)
             for e in range(E_LOCAL)], axis=-1)
        ew_bf_ref[:, :, :] = ew_ref[:, :, :].astype(jnp.bfloat16)

        barrier = pltpu.get_barrier_semaphore()
        for o in range(1, PLANE):
            wp = lax.rem(my_w + o, PLANE)
            pl.semaphore_signal(barrier, inc=1, device_id=(my_z * PLANE + wp,),
                                device_id_type=pl.DeviceIdType.MESH)
        for o in range(1, N_Z):
            zq = lax.rem(my_z + o, N_Z)
            pl.semaphore_signal(barrier, inc=1, device_id=(zq * PLANE + my_w,),
                                device_id_type=pl.DeviceIdType.MESH)
        pl.semaphore_wait(barrier, 6)

        sends = []

        BLK = PLANE * ROWS
        for k in range(1, N_Z + 1):
            zk = lax.rem(my_z + k, N_Z)
            rs = zk * BLK
            xb = x_ref[pl.ds(rs, BLK), :]
            acc = jnp.dot((xb * scale_ref[pl.ds(rs, BLK), 0:1]).astype(jnp.bfloat16),
                          ew_bf_ref[0], preferred_element_type=jnp.float32)
            for e in range(1, E_LOCAL):
                acc += jnp.dot((xb * scale_ref[pl.ds(rs, BLK), e:e + 1]).astype(jnp.bfloat16),
                               ew_bf_ref[e], preferred_element_type=jnp.float32)
            partial_ref[pl.ds(rs, BLK), :] = acc
            partial_bf_ref[pl.ds(rs, BLK), :] = acc.astype(jnp.bfloat16)
            for o in range(1, PLANE):
                wp = lax.rem(my_w + o, PLANE)
                dest = my_z * PLANE + wp
                chunk = 4 * zk + wp
                rdma = pltpu.make_async_remote_copy(
                    src_ref=partial_bf_ref.at[pl.ds(chunk * ROWS, ROWS), :],
                    dst_ref=rp1_ref.at[my_w, zk],
                    send_sem=p1_send_sems.at[wp, zk],
                    recv_sem=p1_recv_sems.at[my_w, zk],
                    device_id=(dest,),
                    device_id_type=pl.DeviceIdType.MESH,
                )
                rdma.start()
                sends.append(rdma)

        def accum_zgroup(zk):
            chunk = 4 * zk + my_w
            acc2 = partial_ref[pl.ds(chunk * ROWS, ROWS), :]
            for oo in range(1, PLANE):
                wq = lax.rem(my_w + oo, PLANE)
                recv = pltpu.make_async_remote_copy(
                    src_ref=partial_bf_ref.at[pl.ds(0, ROWS), :],
                    dst_ref=rp1_ref.at[wq, zk],
                    send_sem=p1_send_sems.at[wq, zk],
                    recv_sem=p1_recv_sems.at[wq, zk],
                    device_id=(d,),
                    device_id_type=pl.DeviceIdType.MESH,
                )
                recv.wait_recv()
                acc2 = acc2 + rp1_ref[wq, zk].astype(jnp.float32)
            return acc2

        for o in range(1, N_Z):
            zq = lax.rem(my_z + o, N_Z)
            dest = zq * PLANE + my_w
            acc2 = accum_zgroup(zq)
            partial_bf_ref[pl.ds(dest * ROWS, ROWS), :] = acc2.astype(jnp.bfloat16)
            rdma = pltpu.make_async_remote_copy(
                src_ref=partial_bf_ref.at[pl.ds(dest * ROWS, ROWS), :],
                dst_ref=rp2_ref.at[my_z],
                send_sem=p2_send_sems.at[zq],
                recv_sem=p2_recv_sems.at[my_z],
                device_id=(dest,),
                device_id_type=pl.DeviceIdType.MESH,
            )
            rdma.start()
            sends.append(rdma)

        result = accum_zgroup(my_z)
        result += x_ref[pl.ds(d * ROWS, ROWS), :] @ sw_ref[:, :]

        for o in range(1, N_Z):
            zr = lax.rem(my_z + o, N_Z)
            recv = pltpu.make_async_remote_copy(
                src_ref=partial_bf_ref.at[pl.ds(0, ROWS), :],
                dst_ref=rp2_ref.at[zr],
                send_sem=p2_send_sems.at[zr],
                recv_sem=p2_recv_sems.at[zr],
                device_id=(d,),
                device_id_type=pl.DeviceIdType.MESH,
            )
            recv.wait_recv()
            result += rp2_ref[zr].astype(jnp.float32)
        out_ref[:, :] = result

        for rdma in sends:
            rdma.wait_send()

    return pl.pallas_call(
        body,
        out_shape=jax.ShapeDtypeStruct((ROWS, D_OUT), jnp.float32),
        in_specs=[pl.BlockSpec(memory_space=pltpu.VMEM)] * 5,
        out_specs=pl.BlockSpec(memory_space=pltpu.VMEM),
        scratch_shapes=[
            pltpu.VMEM((N_TOK, D_OUT), jnp.float32),
            pltpu.VMEM((N_TOK, D_OUT), jnp.bfloat16),
            pltpu.VMEM((PLANE, N_Z, ROWS, D_OUT), jnp.bfloat16),
            pltpu.VMEM((N_Z, ROWS, D_OUT), jnp.bfloat16),
            pltpu.VMEM((N_TOK, E_LOCAL), jnp.float32),
            pltpu.VMEM((E_LOCAL, D_IN, D_OUT), jnp.bfloat16),
            pltpu.SemaphoreType.DMA((PLANE, N_Z)),
            pltpu.SemaphoreType.DMA((PLANE, N_Z)),
            pltpu.SemaphoreType.DMA((N_Z,)),
            pltpu.SemaphoreType.DMA((N_Z,)),
        ],
        compiler_params=pltpu.CompilerParams(collective_id=0),
    )(x, router_W, route_idx, expert_W, shared_W)


# device time: 16325 ns/iter; 2.1414x vs baseline; 2.1414x over previous
import jax
import jax.numpy as jnp
from jax import lax
from jax.experimental import pallas as pl
from jax.experimental.pallas import tpu as pltpu

N_DEV = 16
PLANE = 4
N_Z = 4
N_TOK = 1024
D_IN = 512
D_OUT = 1024
E_LOCAL = 4
ROWS = N_TOK // N_DEV


def kernel(x, router_W, route_idx, expert_W, shared_W):
    def body(x_ref, rw_ref, ri_ref, ew_ref, sw_ref, out_ref,
             partial_ref, partial_bf_ref, rp1_ref, rp2_ref, scale_ref,
             p1_send_sems, p1_recv_sems, p2_send_sems, p2_recv_sems):
        d = lax.axis_index("i")
        my_z = lax.div(d, PLANE)
        my_w = lax.rem(d, PLANE)

        scores = x_ref[:, :] @ rw_ref[:, :]
        m = jnp.max(scores, axis=-1, keepdims=True)
        p = jnp.exp(scores - m)
        probs = p / jnp.sum(p, axis=-1, keepdims=True)
        route = ri_ref[:, :]
        eids = lax.broadcasted_iota(route.dtype, scores.shape, 1)
        coef = jnp.sum(jnp.where(eids == route, probs, 0.0),
                       axis=-1, keepdims=True)
        scale_ref[:, :] = jnp.concatenate(
            [jnp.where(route == d * E_LOCAL + e, coef, 0.0)
             for e in range(E_LOCAL)], axis=-1)

        barrier = pltpu.get_barrier_semaphore()
        for o in range(1, PLANE):
            wp = lax.rem(my_w + o, PLANE)
            pl.semaphore_signal(barrier, inc=1, device_id=(my_z * PLANE + wp,),
                                device_id_type=pl.DeviceIdType.MESH)
        for o in range(1, N_Z):
            zq = lax.rem(my_z + o, N_Z)
            pl.semaphore_signal(barrier, inc=1, device_id=(zq * PLANE + my_w,),
                                device_id_type=pl.DeviceIdType.MESH)
        pl.semaphore_wait(barrier, 6)

        sends = []

        BLK = PLANE * ROWS
        for k in range(1, N_Z + 1):
            zk = lax.rem(my_z + k, N_Z)
            rs = zk * BLK
            xb = x_ref[pl.ds(rs, BLK), :]
            acc = (xb * scale_ref[pl.ds(rs, BLK), 0:1]) @ ew_ref[0]
            for e in range(1, E_LOCAL):
                acc += (xb * scale_ref[pl.ds(rs, BLK), e:e + 1]) @ ew_ref[e]
            partial_ref[pl.ds(rs, BLK), :] = acc
            partial_bf_ref[pl.ds(rs, BLK), :] = acc.astype(jnp.bfloat16)
            for o in range(1, PLANE):
                wp = lax.rem(my_w + o, PLANE)
                dest = my_z * PLANE + wp
                chunk = 4 * zk + wp
                rdma = pltpu.make_async_remote_copy(
                    src_ref=partial_bf_ref.at[pl.ds(chunk * ROWS, ROWS), :],
                    dst_ref=rp1_ref.at[my_w, zk],
                    send_sem=p1_send_sems.at[wp, zk],
                    recv_sem=p1_recv_sems.at[my_w, zk],
                    device_id=(dest,),
                    device_id_type=pl.DeviceIdType.MESH,
                )
                rdma.start()
                sends.append(rdma)

        def accum_zgroup(zk):
            chunk = 4 * zk + my_w
            acc2 = partial_ref[pl.ds(chunk * ROWS, ROWS), :]
            for oo in range(1, PLANE):
                wq = lax.rem(my_w + oo, PLANE)
                recv = pltpu.make_async_remote_copy(
                    src_ref=partial_bf_ref.at[pl.ds(0, ROWS), :],
                    dst_ref=rp1_ref.at[wq, zk],
                    send_sem=p1_send_sems.at[wq, zk],
                    recv_sem=p1_recv_sems.at[wq, zk],
                    device_id=(d,),
                    device_id_type=pl.DeviceIdType.MESH,
                )
                recv.wait_recv()
                acc2 = acc2 + rp1_ref[wq, zk].astype(jnp.float32)
            return acc2

        for o in range(1, N_Z):
            zq = lax.rem(my_z + o, N_Z)
            dest = zq * PLANE + my_w
            acc2 = accum_zgroup(zq)
            partial_bf_ref[pl.ds(dest * ROWS, ROWS), :] = acc2.astype(jnp.bfloat16)
            rdma = pltpu.make_async_remote_copy(
                src_ref=partial_bf_ref.at[pl.ds(dest * ROWS, ROWS), :],
                dst_ref=rp2_ref.at[my_z],
                send_sem=p2_send_sems.at[zq],
                recv_sem=p2_recv_sems.at[my_z],
                device_id=(dest,),
                device_id_type=pl.DeviceIdType.MESH,
            )
            rdma.start()
            sends.append(rdma)

        result = accum_zgroup(my_z)
        result += x_ref[pl.ds(d * ROWS, ROWS), :] @ sw_ref[:, :]

        for o in range(1, N_Z):
            zr = lax.rem(my_z + o, N_Z)
            recv = pltpu.make_async_remote_copy(
                src_ref=partial_bf_ref.at[pl.ds(0, ROWS), :],
                dst_ref=rp2_ref.at[zr],
                send_sem=p2_send_sems.at[zr],
                recv_sem=p2_recv_sems.at[zr],
                device_id=(d,),
                device_id_type=pl.DeviceIdType.MESH,
            )
            recv.wait_recv()
            result += rp2_ref[zr].astype(jnp.float32)
        out_ref[:, :] = result

        for rdma in sends:
            rdma.wait_send()

    return pl.pallas_call(
        body,
        out_shape=jax.ShapeDtypeStruct((ROWS, D_OUT), jnp.float32),
        in_specs=[pl.BlockSpec(memory_space=pltpu.VMEM)] * 5,
        out_specs=pl.BlockSpec(memory_space=pltpu.VMEM),
        scratch_shapes=[
            pltpu.VMEM((N_TOK, D_OUT), jnp.float32),
            pltpu.VMEM((N_TOK, D_OUT), jnp.bfloat16),
            pltpu.VMEM((PLANE, N_Z, ROWS, D_OUT), jnp.bfloat16),
            pltpu.VMEM((N_Z, ROWS, D_OUT), jnp.bfloat16),
            pltpu.VMEM((N_TOK, E_LOCAL), jnp.float32),
            pltpu.SemaphoreType.DMA((PLANE, N_Z)),
            pltpu.SemaphoreType.DMA((PLANE, N_Z)),
            pltpu.SemaphoreType.DMA((N_Z,)),
            pltpu.SemaphoreType.DMA((N_Z,)),
        ],
        compiler_params=pltpu.CompilerParams(collective_id=0),
    )(x, router_W, route_idx, expert_W, shared_W)
